# baseline (speedup 1.0000x reference)
import functools

import jax
import jax.numpy as jnp
from jax import lax
from jax.experimental import pallas as pl
from jax.experimental.pallas import tpu as pltpu
from jax.experimental.pallas import tpu_sc as plsc

B = 4096
L = 200
D = 64
HID = 128

NW = 32
BPW = B // NW
G = 4
NCH = BPW // G
W = 80
NGATHER = (G * L) // W
LANES = 16
NCOL = D // LANES
RUNROLL = 4


def _sc_pool_body(idx_hbm, table_hbm, out_hbm, idx_v, rows_v, acc_v, sem):
    wid = lax.axis_index("s") * 2 + lax.axis_index("c")
    base_row = wid * BPW

    @pl.loop(0, NCH)
    def _chunk(ci):
        row0 = base_row + ci * G
        pltpu.sync_copy(idx_hbm.at[pl.ds(row0 * L, G * L)], idx_v)
        copies = []
        for k in range(NGATHER):
            copies.append(
                pltpu.async_copy(
                    table_hbm.at[idx_v.at[pl.ds(k * W, W)]],
                    rows_v.at[pl.ds(k * W, W)],
                    sem,
                )
            )
        for c in copies:
            c.wait()
        for g in range(G):
            def body(r, carry, g=g):
                out = []
                for c in range(NCOL):
                    a = carry[c]
                    for u in range(RUNROLL):
                        a = a + rows_v[g * L + r * RUNROLL + u,
                                       pl.ds(c * LANES, LANES)]
                    out.append(a)
                return tuple(out)

            zero = jnp.zeros((LANES,), jnp.float32)
            acc = lax.fori_loop(0, L // RUNROLL, body, (zero,) * NCOL)
            for c in range(NCOL):
                acc_v[g, pl.ds(c * LANES, LANES)] = acc[c]
        pltpu.sync_copy(acc_v, out_hbm.at[pl.ds(row0, G)])


def _sc_pool(idx_flat, emb_table):
    mesh = plsc.VectorSubcoreMesh(core_axis_name="c", subcore_axis_name="s")
    kern = pl.kernel(
        _sc_pool_body,
        out_type=jax.ShapeDtypeStruct((B, D), jnp.float32),
        mesh=mesh,
        scratch_types=[
            pltpu.VMEM((G * L,), jnp.int32),
            pltpu.VMEM((G * L, D), jnp.float32),
            pltpu.VMEM((G, D), jnp.float32),
            pltpu.SemaphoreType.DMA,
        ],
        compiler_params=pltpu.CompilerParams(use_tc_tiling_on_sc=False),
    )
    return kern(idx_flat, emb_table)


def _mlp_body(x_ref, w1_ref, b1_ref, w2_ref, b2_ref, o_ref):
    h = lax.dot_general(
        x_ref[...], w1_ref[...], (((1,), (1,)), ((), ())),
        preferred_element_type=jnp.float32,
    )
    h = jnp.tanh(h + b1_ref[...])
    o_ref[...] = jnp.sum(h * w2_ref[...], axis=1, keepdims=True) + b2_ref[...]


def kernel(x, emb_table, W1, b1, W2, b2):
    idx_flat = x.reshape(-1)
    pooled = _sc_pool(idx_flat, emb_table)
    out = pl.pallas_call(
        _mlp_body,
        out_shape=jax.ShapeDtypeStruct((B, 1), jnp.float32),
    )(pooled, W1, b1.reshape(1, HID), W2, b2.reshape(1, 1))
    return out
